# Initial kernel scaffold; baseline (speedup 1.0000x reference)
import jax
import jax.numpy as jnp
from jax.experimental import pallas as pl


def kernel(inputs, gamma, beta, W, b):
    raise NotImplementedError("write your pallas kernel here")


# speedup vs baseline: 8.1828x; 8.1828x over previous
import functools

import jax
import jax.numpy as jnp
import numpy as np
from jax import lax
from jax.experimental import pallas as pl
from jax.experimental.pallas import tpu as pltpu

_K_CLUSTERS = 512
_KNN_K = 5
_CHUNK = 512


def _lane_sum_tree(v):
    c = v.shape[-1]
    acc = v[:, 0:8]
    for t in range(1, c // 8):
        acc = acc + v[:, 8 * t:8 * (t + 1)]
    acc = acc[:, 0:4] + acc[:, 4:8]
    acc = acc[:, 0:2] + acc[:, 2:4]
    return acc[:, 0:1] + acc[:, 1:2]


def _stage1_body(in_ref, g_ref, b_ref, w_ref, bb_ref, noise_ref,
                 xn_ref, wtok_ref, sq_ref, score_ref,
                 dens_ref, dmax_ref):
    n = in_ref.shape[1]
    c = in_ref.shape[2]
    chunk = min(_CHUNK, n)
    nchunks = n // chunk
    x_raw = in_ref[0]
    mu = _lane_sum_tree(x_raw) * (1.0 / c)
    var = _lane_sum_tree((x_raw - mu) ** 2) * (1.0 / c)
    xn = (x_raw - mu) / jnp.sqrt(var + 1e-5) * g_ref[:] + b_ref[:]
    xn_ref[0] = xn
    tok_score = lax.dot_general(
        xn, w_ref[:], (((1,), (0,)), ((), ())),
        preferred_element_type=jnp.float32,
        precision=lax.Precision.HIGHEST) + bb_ref[:]
    wtok = jnp.exp(tok_score)
    wtok_ref[0, 0] = wtok[:, 0]
    sq = _lane_sum_tree(xn * xn)[:, 0]
    sq_ref[0, 0] = sq
    sqrt_c = jnp.sqrt(jnp.asarray(c, dtype=jnp.float32))
    inf = jnp.float32(jnp.inf)

    for r in range(nchunks):
        rows = xn[r * chunk:(r + 1) * chunk]
        sq_r = sq[r * chunk:(r + 1) * chunk]
        prod = lax.dot_general(
            rows, xn, (((1,), (1,)), ((), ())),
            preferred_element_type=jnp.float32)
        d2 = sq_r[:, None] + sq[None, :] - 2.0 * prod
        dist = jnp.sqrt(jnp.maximum(d2, 0.0)) / sqrt_c
        cmax = jnp.max(dist)
        if r == 0:
            dmax_ref[0, 0] = cmax
        else:
            dmax_ref[0, 0] = jnp.maximum(dmax_ref[0, 0], cmax)
        iota_j = lax.broadcasted_iota(jnp.int32, (chunk, n), 1)
        work = dist
        ms = []
        for _ in range(_KNN_K):
            m = jnp.min(work, axis=1)
            ms.append(m * m)
            pos = jnp.min(
                jnp.where(work == m[:, None], iota_j, n), axis=1)
            work = jnp.where(iota_j == pos[:, None], inf, work)
        mean5 = (((ms[0] + ms[4]) + ms[2]) + (ms[1] + ms[3])) * np.float32(0.2)
        exp_arg = jnp.clip(-mean5, -10.0, 10.0)
        dens = jnp.exp(exp_arg) + noise_ref[0, 0, r * chunk:(r + 1) * chunk]
        dens_ref[0, pl.ds(r * chunk, chunk)] = dens

    dens_all = dens_ref[0]
    dmax = dmax_ref[0, 0]
    for r in range(nchunks):
        rows = xn[r * chunk:(r + 1) * chunk]
        sq_r = sq[r * chunk:(r + 1) * chunk]
        dens_r = dens_all[r * chunk:(r + 1) * chunk]
        prod = lax.dot_general(
            rows, xn, (((1,), (1,)), ((), ())),
            preferred_element_type=jnp.float32)
        d2 = sq_r[:, None] + sq[None, :] - 2.0 * prod
        dist = jnp.sqrt(jnp.maximum(d2, 0.0)) / sqrt_c
        masked = jnp.where(dens_all[None, :] > dens_r[:, None], dist, inf)
        mn = jnp.min(masked, axis=1)
        dist_i = jnp.minimum(mn, dmax)
        score_ref[0, 0, pl.ds(r * chunk, chunk)] = dist_i * dens_r


def _stage3_body(xn_ref, sq_ref, wtok_ref, score_ref, out_ref):
    n = xn_ref.shape[1]
    c = xn_ref.shape[2]
    k_sel = out_ref.shape[1]
    chunk = min(_CHUNK, n)
    nchunks = n // chunk
    xn = xn_ref[0]
    sq = sq_ref[0, 0]
    wtok = wtok_ref[0, 0]
    score = score_ref[0, 0]
    sqrt_c = jnp.sqrt(jnp.asarray(c, dtype=jnp.float32))

    idx_n = lax.broadcasted_iota(jnp.int32, (chunk, n), 1)
    cnt = jnp.zeros((n,), dtype=jnp.float32)
    for r in range(nchunks):
        s_m = score[r * chunk:(r + 1) * chunk]
        idx_m = lax.broadcasted_iota(
            jnp.int32, (chunk, n), 0) + r * chunk
        gt = (s_m[:, None] > score[None, :]) | (
            (s_m[:, None] == score[None, :]) & (idx_m < idx_n))
        cnt = cnt + jnp.sum(gt.astype(jnp.float32), axis=0)
    rank = cnt.astype(jnp.int32)

    iota_k = lax.broadcasted_iota(jnp.int32, (k_sel, n), 0)
    e_sel = (rank[None, :] == iota_k).astype(jnp.float32)
    sel = lax.dot_general(
        e_sel, xn, (((1,), (0,)), ((), ())),
        preferred_element_type=jnp.float32,
        precision=lax.Precision.HIGHEST)
    sq_sel = lax.dot_general(
        e_sel, sq[:, None], (((1,), (0,)), ((), ())),
        preferred_element_type=jnp.float32,
        precision=lax.Precision.HIGHEST)
    prod = lax.dot_general(
        sel, xn, (((1,), (1,)), ((), ())),
        preferred_element_type=jnp.float32)
    d2s = sq_sel + sq[None, :] - 2.0 * prod
    dists = jnp.sqrt(jnp.maximum(d2s, 0.0)) / sqrt_c

    mnv = jnp.min(dists, axis=0)
    cl = jnp.min(jnp.where(dists == mnv[None, :], iota_k, k_sel), axis=0)
    cl = jnp.where(rank < k_sel, rank, cl)

    at_mat = (cl[None, :] == iota_k).astype(jnp.float32)
    aw = lax.dot_general(
        at_mat, wtok[:, None], (((1,), (0,)), ((), ())),
        preferred_element_type=jnp.float32,
        precision=lax.Precision.HIGHEST) + 1e-6
    iota_kc = lax.broadcasted_iota(jnp.int32, (n, k_sel), 1)
    a_mat = (cl[:, None] == iota_kc).astype(jnp.float32)
    denom = lax.dot_general(
        a_mat, aw, (((1,), (0,)), ((), ())),
        preferred_element_type=jnp.float32,
        precision=lax.Precision.HIGHEST)
    src = xn * (wtok[:, None] / denom)
    out_ref[0] = lax.dot_general(
        at_mat, src, (((1,), (0,)), ((), ())),
        preferred_element_type=jnp.float32,
        precision=lax.Precision.HIGHEST)


@jax.jit
def kernel(inputs, gamma, beta, W, b):
    bsz, n, c = inputs.shape
    k_sel = _K_CLUSTERS
    noise = (jax.random.uniform(jax.random.key(12345), (bsz, n),
                                dtype=jnp.float32) * 1e-6)
    noise3 = noise.reshape(bsz, 1, n)

    f32 = jnp.float32
    xn, wtok, sq, score = pl.pallas_call(
        _stage1_body,
        grid=(bsz,),
        in_specs=[
            pl.BlockSpec((1, n, c), lambda i: (i, 0, 0)),
            pl.BlockSpec((c,), lambda i: (0,)),
            pl.BlockSpec((c,), lambda i: (0,)),
            pl.BlockSpec((c, 1), lambda i: (0, 0)),
            pl.BlockSpec((1,), lambda i: (0,)),
            pl.BlockSpec((1, 1, n), lambda i: (i, 0, 0)),
        ],
        out_specs=[
            pl.BlockSpec((1, n, c), lambda i: (i, 0, 0)),
            pl.BlockSpec((1, 1, n), lambda i: (i, 0, 0)),
            pl.BlockSpec((1, 1, n), lambda i: (i, 0, 0)),
            pl.BlockSpec((1, 1, n), lambda i: (i, 0, 0)),
        ],
        out_shape=[
            jax.ShapeDtypeStruct((bsz, n, c), f32),
            jax.ShapeDtypeStruct((bsz, 1, n), f32),
            jax.ShapeDtypeStruct((bsz, 1, n), f32),
            jax.ShapeDtypeStruct((bsz, 1, n), f32),
        ],
        scratch_shapes=[
            pltpu.VMEM((1, n), f32),
            pltpu.SMEM((1, 1), f32),
        ],
    )(inputs, gamma, beta, W, b, noise3)

    out = pl.pallas_call(
        _stage3_body,
        grid=(bsz,),
        in_specs=[
            pl.BlockSpec((1, n, c), lambda i: (i, 0, 0)),
            pl.BlockSpec((1, 1, n), lambda i: (i, 0, 0)),
            pl.BlockSpec((1, 1, n), lambda i: (i, 0, 0)),
            pl.BlockSpec((1, 1, n), lambda i: (i, 0, 0)),
        ],
        out_specs=pl.BlockSpec((1, k_sel, c), lambda i: (i, 0, 0)),
        out_shape=jax.ShapeDtypeStruct((bsz, k_sel, c), f32),
    )(xn, sq, wtok, score)
    return out


# speedup vs baseline: 9.1504x; 1.1182x over previous
import functools

import jax
import jax.numpy as jnp
import numpy as np
from jax import lax
from jax.experimental import pallas as pl
from jax.experimental.pallas import tpu as pltpu

_K_CLUSTERS = 512
_KNN_K = 5
_CHUNK = 512


def _lane_sum_tree(v):
    c = v.shape[-1]
    acc = v[:, 0:8]
    for t in range(1, c // 8):
        acc = acc + v[:, 8 * t:8 * (t + 1)]
    acc = acc[:, 0:4] + acc[:, 4:8]
    acc = acc[:, 0:2] + acc[:, 2:4]
    return acc[:, 0:1] + acc[:, 1:2]


def _body(in_ref, g_ref, b_ref, w_ref, bb_ref, noise_ref, out_ref,
          dist_ref, dens_ref, score_ref, dmax_ref):
    n = in_ref.shape[1]
    c = in_ref.shape[2]
    k_sel = out_ref.shape[1]
    chunk = min(_CHUNK, n)
    nchunks = n // chunk
    x_raw = in_ref[0]
    mu = _lane_sum_tree(x_raw) * (1.0 / c)
    var = _lane_sum_tree((x_raw - mu) ** 2) * (1.0 / c)
    xn = (x_raw - mu) / jnp.sqrt(var + 1e-5) * g_ref[:] + b_ref[:]
    tok_score = lax.dot_general(
        xn, w_ref[:], (((1,), (0,)), ((), ())),
        preferred_element_type=jnp.float32,
        precision=lax.Precision.HIGHEST) + bb_ref[:]
    wtok = jnp.exp(tok_score)[:, 0]
    sq = _lane_sum_tree(xn * xn)[:, 0]
    sqrt_c = jnp.sqrt(jnp.asarray(c, dtype=jnp.float32))
    inf = jnp.float32(jnp.inf)
    iota_cj = lax.broadcasted_iota(jnp.int32, (chunk, n), 1)

    for r in range(nchunks):
        rows = xn[r * chunk:(r + 1) * chunk]
        sq_r = sq[r * chunk:(r + 1) * chunk]
        prod = lax.dot_general(
            rows, xn, (((1,), (1,)), ((), ())),
            preferred_element_type=jnp.float32)
        d2 = sq_r[:, None] + sq[None, :] - 2.0 * prod
        dist = jnp.sqrt(jnp.maximum(d2, 0.0)) / sqrt_c
        dist_ref[pl.ds(r * chunk, chunk), :] = dist
        cmax = jnp.max(dist)
        if r == 0:
            dmax_ref[0, 0] = cmax
        else:
            dmax_ref[0, 0] = jnp.maximum(dmax_ref[0, 0], cmax)
        work = dist
        ms = []
        for _ in range(_KNN_K):
            m = jnp.min(work, axis=1)
            ms.append(m * m)
            pos = jnp.min(
                jnp.where(work == m[:, None], iota_cj, n), axis=1)
            work = jnp.where(iota_cj == pos[:, None], inf, work)
        mean5 = (((ms[0] + ms[4]) + ms[2]) + (ms[1] + ms[3])) * np.float32(0.2)
        exp_arg = jnp.clip(-mean5, -10.0, 10.0)
        dens = jnp.exp(exp_arg) + noise_ref[0, 0, r * chunk:(r + 1) * chunk]
        dens_ref[0, pl.ds(r * chunk, chunk)] = dens

    dens_all = dens_ref[0]
    dmax = dmax_ref[0, 0]
    for r in range(nchunks):
        dist = dist_ref[pl.ds(r * chunk, chunk), :]
        dens_r = dens_all[r * chunk:(r + 1) * chunk]
        masked = jnp.where(dens_all[None, :] > dens_r[:, None], dist, inf)
        mn = jnp.min(masked, axis=1)
        dist_i = jnp.minimum(mn, dmax)
        score_ref[0, pl.ds(r * chunk, chunk)] = dist_i * dens_r

    score = score_ref[0]
    cnt = jnp.zeros((n,), dtype=jnp.float32)
    for r in range(nchunks):
        s_m = score[r * chunk:(r + 1) * chunk]
        idx_m = lax.broadcasted_iota(jnp.int32, (chunk, n), 0) + r * chunk
        gt = (s_m[:, None] > score[None, :]) | (
            (s_m[:, None] == score[None, :]) & (idx_m < iota_cj))
        cnt = cnt + jnp.sum(gt.astype(jnp.float32), axis=0)
    rank = cnt.astype(jnp.int32)

    iota_k = lax.broadcasted_iota(jnp.int32, (k_sel, n), 0)
    e_sel = (rank[None, :] == iota_k).astype(jnp.float32)
    sel = lax.dot_general(
        e_sel, xn, (((1,), (0,)), ((), ())),
        preferred_element_type=jnp.float32,
        precision=lax.Precision.HIGHEST)
    sq_sel = lax.dot_general(
        e_sel, sq[:, None], (((1,), (0,)), ((), ())),
        preferred_element_type=jnp.float32,
        precision=lax.Precision.HIGHEST)
    prod_s = lax.dot_general(
        sel, xn, (((1,), (1,)), ((), ())),
        preferred_element_type=jnp.float32)
    d2s = sq_sel + sq[None, :] - 2.0 * prod_s
    dists = jnp.sqrt(jnp.maximum(d2s, 0.0)) / sqrt_c

    mnv = jnp.min(dists, axis=0)
    cl = jnp.min(jnp.where(dists == mnv[None, :], iota_k, k_sel), axis=0)
    cl = jnp.where(rank < k_sel, rank, cl)

    at_mat = (cl[None, :] == iota_k).astype(jnp.float32)
    aw = lax.dot_general(
        at_mat, wtok[:, None], (((1,), (0,)), ((), ())),
        preferred_element_type=jnp.float32,
        precision=lax.Precision.HIGHEST) + 1e-6
    iota_kc = lax.broadcasted_iota(jnp.int32, (n, k_sel), 1)
    a_mat = (cl[:, None] == iota_kc).astype(jnp.float32)
    denom = lax.dot_general(
        a_mat, aw, (((1,), (0,)), ((), ())),
        preferred_element_type=jnp.float32,
        precision=lax.Precision.HIGHEST)
    src = xn * (wtok[:, None] / denom)
    out_ref[0] = lax.dot_general(
        at_mat, src, (((1,), (0,)), ((), ())),
        preferred_element_type=jnp.float32,
        precision=lax.Precision.HIGHEST)


@jax.jit
def kernel(inputs, gamma, beta, W, b):
    bsz, n, c = inputs.shape
    k_sel = _K_CLUSTERS
    noise = (jax.random.uniform(jax.random.key(12345), (bsz, n),
                                dtype=jnp.float32) * 1e-6)
    noise3 = noise.reshape(bsz, 1, n)

    f32 = jnp.float32
    out = pl.pallas_call(
        _body,
        grid=(bsz,),
        in_specs=[
            pl.BlockSpec((1, n, c), lambda i: (i, 0, 0)),
            pl.BlockSpec((c,), lambda i: (0,)),
            pl.BlockSpec((c,), lambda i: (0,)),
            pl.BlockSpec((c, 1), lambda i: (0, 0)),
            pl.BlockSpec((1,), lambda i: (0,)),
            pl.BlockSpec((1, 1, n), lambda i: (i, 0, 0)),
        ],
        out_specs=pl.BlockSpec((1, k_sel, c), lambda i: (i, 0, 0)),
        out_shape=jax.ShapeDtypeStruct((bsz, k_sel, c), f32),
        scratch_shapes=[
            pltpu.VMEM((n, n), f32),
            pltpu.VMEM((1, n), f32),
            pltpu.VMEM((1, n), f32),
            pltpu.SMEM((1, 1), f32),
        ],
    )(inputs, gamma, beta, W, b, noise3)
    return out
